# baseline (speedup 1.0000x reference)
import jax
import jax.numpy as jnp
import numpy as np
from jax import lax
from jax.experimental import pallas as pl
from jax.experimental.pallas import tpu as pltpu
from jax.experimental.shard_map import shard_map
from jax.sharding import Mesh, PartitionSpec as P

H = 64
H2 = 2 * H
KW = 8
C_IN = 2
L_IN = 128
XW = C_IN * L_IN
T_CONV = L_IN - KW + 1
T = T_CONV // 2
NC = 11
NCP = 128
B_TILE = 1024


def _make_body(bh):

    def body(x_ref, wconv_ref, bconv_ref, w1_ref, b1_ref,
             w2_ref, b2_ref, wfc_ref, bfc_ref,
             out_ref, hseq_ref):
        f32 = jnp.float32

        bconv = bconv_ref[...]

        def conv_step(t):
            d = jnp.dot(x_ref[0], wconv_ref[t], preferred_element_type=f32)
            ce = jnp.maximum(d[:, 0:H2] + bconv, 0.0)
            co = jnp.maximum(d[:, H2:2 * H2] + bconv, 0.0)
            return jnp.maximum(ce, co)

        w1 = w1_ref[...]
        bb1 = b1_ref[...]
        w2 = w2_ref[...]
        bb2 = b2_ref[...]
        bfc = bfc_ref[...]
        zeros = jnp.zeros((bh, H2), f32)

        def cell(gates, c):
            sig = 0.5 * jnp.tanh(gates[:, 0:3 * H2]) + 0.5
            g = jnp.tanh(gates[:, 3 * H2:4 * H2])
            c = sig[:, H2:2 * H2] * c + sig[:, 0:H2] * g
            h = sig[:, 2 * H2:3 * H2] * jnp.tanh(c)
            return h, c

        def step1(pooled, h1, c1):
            a = jnp.concatenate([pooled, h1], axis=1)
            gates = jnp.dot(a, w1, preferred_element_type=f32) + bb1
            return cell(gates, c1)

        def step2(h1_prev, h2, c2):
            a = jnp.concatenate([h1_prev, h2], axis=1)
            gates = jnp.dot(a, w2, preferred_element_type=f32) + bb2
            return cell(gates, c2)

        h1, c1 = step1(conv_step(0), zeros, zeros)
        p = conv_step(1)

        def fused(t, carry):
            p, h1, c1, h2, c2 = carry
            h1n, c1n = step1(p, h1, c1)
            h2n, c2n = step2(h1, h2, c2)
            pn = conv_step(t + 1)
            r0 = pl.multiple_of((t - 1) * bh, bh)
            hseq_ref[pl.ds(r0, bh), :] = h2n.astype(jnp.bfloat16)
            return (pn, h1n, c1n, h2n, c2n)

        p, h1, c1, h2, c2 = lax.fori_loop(
            1, T, fused, (p, h1, c1, zeros, zeros), unroll=4)

        h2, c2 = step2(h1, h2, c2)
        hseq_ref[pl.ds((T - 1) * bh, bh), :] = h2.astype(jnp.bfloat16)

        def fc_step(t, acc):
            r0 = pl.multiple_of(t * bh, bh)
            return acc + jnp.dot(hseq_ref[pl.ds(r0, bh), :], wfc_ref[t],
                                 preferred_element_type=f32)

        acc = lax.fori_loop(0, T, fc_step,
                            jnp.zeros((bh, 2 * NCP), f32), unroll=4)

        for s in range(2):
            logits = acc[:, s * NCP:(s + 1) * NCP] + bfc
            m = jnp.max(logits, axis=1, keepdims=True)
            e = jnp.exp(logits - m)
            out_ref[0, pl.ds(s * bh, bh), :] = e / jnp.sum(e, axis=1,
                                                           keepdims=True)

    return body


def _gate_sel():
    d = np.zeros((2, 4 * H, 8 * H), np.float32)
    grp_of = {0: 0, 1: 1, 2: 3, 3: 2}
    for a in range(4 * H):
        grp = grp_of[a // H]
        scale = 0.5 if grp < 3 else 1.0
        for s in range(2):
            d[s, a, grp * H2 + s * H + (a % H)] = scale
    return d


_GATE_SEL = _gate_sel()


def _pack_lstm(w_ih, w_hh, b_ih, b_hh):
    dsel = jnp.asarray(_GATE_SEL)
    wstack = jnp.stack([w_ih, w_hh])
    w = jnp.einsum('pab,sac->psbc', wstack, dsel).reshape(2 * H2, 4 * H2)
    bias = jnp.einsum('a,ac->c', b_ih + b_hh,
                      dsel[0] + dsel[1]).reshape(1, 4 * H2)
    return w, bias


def _conv_sel():
    sel = np.zeros((T, L_IN, 2, KW), np.float32)
    for t in range(T):
        for eo in range(2):
            for k in range(KW):
                sel[t, 2 * t + eo + k, eo, k] = 1.0
    return sel


_CONV_SEL = _conv_sel()
_EYE2 = np.eye(2, dtype=np.float32)


def _conv_weights(conv_w):
    wk = conv_w.transpose(1, 2, 0).astype(jnp.float32)
    wtp = jnp.einsum('tlek,ckh,sm->tsclemh',
                     jnp.asarray(_CONV_SEL), wk, jnp.asarray(_EYE2))
    return wtp.reshape(T, 2 * XW, 2 * H2)


def _fc_sel():
    f = np.zeros((2, NC, 2 * NCP), np.float32)
    for s in range(2):
        for n in range(NC):
            f[s, n, s * NCP + n] = 1.0
    return f


_FC_SEL = _fc_sel()


def _core(x2, wconv, bconv, w1, b1, w2, b2, wfc_p, bfc, b):
    f32 = jnp.float32
    bh = b // 2
    G = x2.shape[0]

    def full_spec(a):
        n = a.ndim
        return pl.BlockSpec(a.shape, lambda g, n=n: (0,) * n)

    grid_spec = pltpu.PrefetchScalarGridSpec(
        num_scalar_prefetch=0,
        grid=(G,),
        in_specs=[
            pl.BlockSpec((1, bh, 2 * XW), lambda g: (g, 0, 0)),
            full_spec(wconv), full_spec(bconv),
            full_spec(w1), full_spec(b1),
            full_spec(w2), full_spec(b2),
            full_spec(wfc_p), full_spec(bfc),
        ],
        out_specs=pl.BlockSpec((1, b, NCP), lambda g: (g, 0, 0)),
        scratch_shapes=[
            pltpu.VMEM((T * bh, H2), jnp.bfloat16),
        ],
    )

    return pl.pallas_call(
        _make_body(bh),
        out_shape=jax.ShapeDtypeStruct((G, b, NCP), f32),
        grid_spec=grid_spec,
        compiler_params=pltpu.CompilerParams(dimension_semantics=("parallel",)),
    )(x2, wconv, bconv, w1, b1, w2, b2, wfc_p, bfc)


def _prep_and_run(conv_w, conv_b, w_ih1, w_hh1, b_ih1, b_hh1,
                  w_ih2, w_hh2, b_ih2, b_hh2, fc_w, fc_b, x_bf):
    f32 = jnp.float32
    bf16 = jnp.bfloat16
    B = x_bf.shape[0]
    b = min(B_TILE, 16 * pl.cdiv(B, 16))
    bh = b // 2
    G = pl.cdiv(B, b)
    B_pad = G * b

    xp = jnp.pad(x_bf, ((0, B_pad - B), (0, 0), (0, 0)))
    x2 = (xp.reshape(G, 2, bh, XW).transpose(0, 2, 1, 3)
          .reshape(G, bh, 2 * XW))

    wconv = _conv_weights(conv_w).astype(bf16)
    wconv = jnp.pad(wconv, ((0, 1), (0, 0), (0, 0)))
    bconv = jnp.concatenate([conv_b, conv_b]).reshape(1, H2).astype(f32)
    w1, b1 = _pack_lstm(w_ih1, w_hh1, b_ih1, b_hh1)
    w2, b2 = _pack_lstm(w_ih2, w_hh2, b_ih2, b_hh2)
    wfc_p = jnp.einsum('nth,snc->tshc', fc_w.reshape(NC, T, H),
                       jnp.asarray(_FC_SEL)).reshape(T, H2, 2 * NCP)
    wfc_p = wfc_p.astype(bf16)
    bfc = jnp.concatenate(
        [fc_b.astype(f32), jnp.full((NCP - NC,), -1e30, f32)]).reshape(1, NCP)

    out = _core(x2, wconv, bconv, w1, b1, w2, b2, wfc_p, bfc, b)
    return out.reshape(B_pad, NCP)[:B, :NC]


def kernel(conv_w, conv_b, w_ih1, w_hh1, b_ih1, b_hh1,
           w_ih2, w_hh2, b_ih2, b_hh2, fc_w, fc_b, x):
    args = (conv_w, conv_b, w_ih1, w_hh1, b_ih1, b_hh1,
            w_ih2, w_hh2, b_ih2, b_hh2, fc_w, fc_b)
    x_bf = x.astype(jnp.bfloat16)
    devs = jax.devices()
    B = x.shape[0]
    if len(devs) < 2 or B % 2:
        return _prep_and_run(*args, x_bf)
    mesh = Mesh(np.array(devs[:2]), ("d",))
    f = shard_map(_prep_and_run, mesh=mesh,
                  in_specs=tuple([P()] * 12 + [P("d")]),
                  out_specs=P("d"), check_rep=False)
    if B % 4 == 0:
        half = B // 2
        return jnp.concatenate([f(*args, x_bf[:half]),
                                f(*args, x_bf[half:])], axis=0)
    return f(*args, x_bf)


# speedup vs baseline: 1.0108x; 1.0108x over previous
import jax
import jax.numpy as jnp
import numpy as np
from jax import lax
from jax.experimental import pallas as pl
from jax.experimental.pallas import tpu as pltpu
from jax.experimental.shard_map import shard_map
from jax.sharding import Mesh, PartitionSpec as P

H = 64
H2 = 2 * H
KW = 8
C_IN = 2
L_IN = 128
XW = C_IN * L_IN
T_CONV = L_IN - KW + 1
T = T_CONV // 2
NC = 11
NCP = 128
B_TILE = 1024


def _make_body(bh):

    def body(x_ref, wconv_ref, bconv_ref, w1_ref, b1_ref,
             w2_ref, b2_ref, wfc_ref, bfc_ref,
             out_ref, hseq_ref):
        f32 = jnp.float32

        bconv = bconv_ref[...]

        def conv_step(t):
            d = jnp.dot(x_ref[0], wconv_ref[t], preferred_element_type=f32)
            ce = jnp.maximum(d[:, 0:H2] + bconv, 0.0)
            co = jnp.maximum(d[:, H2:2 * H2] + bconv, 0.0)
            return jnp.maximum(ce, co)

        w1 = w1_ref[...]
        bb1 = b1_ref[...]
        w2 = w2_ref[...]
        bb2 = b2_ref[...]
        bfc = bfc_ref[...]
        zeros = jnp.zeros((bh, H2), f32)

        def cell(gates, c):
            sig = 0.5 * jnp.tanh(gates[:, 0:3 * H2]) + 0.5
            g = jnp.tanh(gates[:, 3 * H2:4 * H2])
            c = sig[:, H2:2 * H2] * c + sig[:, 0:H2] * g
            h = sig[:, 2 * H2:3 * H2] * jnp.tanh(c)
            return h, c

        def step1(pooled, h1, c1):
            a = jnp.concatenate([pooled, h1], axis=1)
            gates = jnp.dot(a, w1, preferred_element_type=f32) + bb1
            return cell(gates, c1)

        def step2(h1_prev, h2, c2):
            a = jnp.concatenate([h1_prev, h2], axis=1)
            gates = jnp.dot(a, w2, preferred_element_type=f32) + bb2
            return cell(gates, c2)

        h1, c1 = step1(conv_step(0), zeros, zeros)
        p = conv_step(1)

        def fused(t, carry):
            p, h1, c1, h2, c2 = carry
            h1n, c1n = step1(p, h1, c1)
            h2n, c2n = step2(h1, h2, c2)
            pn = conv_step(t + 1)
            r0 = pl.multiple_of((t - 1) * bh, bh)
            hseq_ref[pl.ds(r0, bh), :] = h2n.astype(jnp.bfloat16)
            return (pn, h1n, c1n, h2n, c2n)

        p, h1, c1, h2, c2 = lax.fori_loop(
            1, T, fused, (p, h1, c1, zeros, zeros), unroll=4)

        h2, c2 = step2(h1, h2, c2)
        hseq_ref[pl.ds((T - 1) * bh, bh), :] = h2.astype(jnp.bfloat16)

        def fc_step(t, acc):
            r0 = pl.multiple_of(t * bh, bh)
            return acc + jnp.dot(hseq_ref[pl.ds(r0, bh), :], wfc_ref[t],
                                 preferred_element_type=f32)

        acc = lax.fori_loop(0, T, fc_step,
                            jnp.zeros((bh, 2 * NCP), f32), unroll=4)

        for s in range(2):
            logits = acc[:, s * NCP:(s + 1) * NCP] + bfc
            m = jnp.max(logits, axis=1, keepdims=True)
            e = jnp.exp(logits - m)
            out_ref[0, pl.ds(s * bh, bh), :] = e / jnp.sum(e, axis=1,
                                                           keepdims=True)

    return body


def _gate_sel():
    d = np.zeros((2, 4 * H, 8 * H), np.float32)
    grp_of = {0: 0, 1: 1, 2: 3, 3: 2}
    for a in range(4 * H):
        grp = grp_of[a // H]
        scale = 0.5 if grp < 3 else 1.0
        for s in range(2):
            d[s, a, grp * H2 + s * H + (a % H)] = scale
    return d


_GATE_SEL = _gate_sel()


def _pack_lstm(w_ih, w_hh, b_ih, b_hh):
    dsel = jnp.asarray(_GATE_SEL)
    wstack = jnp.stack([w_ih, w_hh])
    w = jnp.einsum('pab,sac->psbc', wstack, dsel).reshape(2 * H2, 4 * H2)
    bias = jnp.einsum('a,ac->c', b_ih + b_hh,
                      dsel[0] + dsel[1]).reshape(1, 4 * H2)
    return w, bias


def _conv_sel():
    sel = np.zeros((T, L_IN, 2, KW), np.float32)
    for t in range(T):
        for eo in range(2):
            for k in range(KW):
                sel[t, 2 * t + eo + k, eo, k] = 1.0
    return sel


_CONV_SEL = _conv_sel()
_EYE2 = np.eye(2, dtype=np.float32)


def _conv_weights(conv_w):
    wk = conv_w.transpose(1, 2, 0).astype(jnp.float32)
    wtp = jnp.einsum('tlek,ckh,sm->tsclemh',
                     jnp.asarray(_CONV_SEL), wk, jnp.asarray(_EYE2))
    return wtp.reshape(T, 2 * XW, 2 * H2)


def _fc_sel():
    f = np.zeros((2, NC, 2 * NCP), np.float32)
    for s in range(2):
        for n in range(NC):
            f[s, n, s * NCP + n] = 1.0
    return f


_FC_SEL = _fc_sel()


def _core(x2, wconv, bconv, w1, b1, w2, b2, wfc_p, bfc, b):
    f32 = jnp.float32
    bh = b // 2
    G = x2.shape[0]

    def full_spec(a):
        n = a.ndim
        return pl.BlockSpec(a.shape, lambda g, n=n: (0,) * n)

    grid_spec = pltpu.PrefetchScalarGridSpec(
        num_scalar_prefetch=0,
        grid=(G,),
        in_specs=[
            pl.BlockSpec((1, bh, 2 * XW), lambda g: (g, 0, 0)),
            full_spec(wconv), full_spec(bconv),
            full_spec(w1), full_spec(b1),
            full_spec(w2), full_spec(b2),
            full_spec(wfc_p), full_spec(bfc),
        ],
        out_specs=pl.BlockSpec((1, b, NCP), lambda g: (g, 0, 0)),
        scratch_shapes=[
            pltpu.VMEM((T * bh, H2), jnp.bfloat16),
        ],
    )

    return pl.pallas_call(
        _make_body(bh),
        out_shape=jax.ShapeDtypeStruct((G, b, NCP), f32),
        grid_spec=grid_spec,
        compiler_params=pltpu.CompilerParams(dimension_semantics=("parallel",)),
    )(x2, wconv, bconv, w1, b1, w2, b2, wfc_p, bfc)


def _prep_and_run(conv_w, conv_b, w_ih1, w_hh1, b_ih1, b_hh1,
                  w_ih2, w_hh2, b_ih2, b_hh2, fc_w, fc_b, x_bf):
    f32 = jnp.float32
    bf16 = jnp.bfloat16
    B = x_bf.shape[0]
    b = min(B_TILE, 16 * pl.cdiv(B, 16))
    bh = b // 2
    G = pl.cdiv(B, b)
    B_pad = G * b

    xp = jnp.pad(x_bf, ((0, B_pad - B), (0, 0), (0, 0)))
    x2 = (xp.reshape(G, 2, bh, XW).transpose(0, 2, 1, 3)
          .reshape(G, bh, 2 * XW))

    wconv = _conv_weights(conv_w).astype(bf16)
    wconv = jnp.pad(wconv, ((0, 1), (0, 0), (0, 0)))
    bconv = jnp.concatenate([conv_b, conv_b]).reshape(1, H2).astype(f32)
    w1, b1 = _pack_lstm(w_ih1, w_hh1, b_ih1, b_hh1)
    w2, b2 = _pack_lstm(w_ih2, w_hh2, b_ih2, b_hh2)
    wfc_p = jnp.einsum('nth,snc->tshc', fc_w.reshape(NC, T, H),
                       jnp.asarray(_FC_SEL)).reshape(T, H2, 2 * NCP)
    wfc_p = wfc_p.astype(bf16)
    bfc = jnp.concatenate(
        [fc_b.astype(f32), jnp.full((NCP - NC,), -1e30, f32)]).reshape(1, NCP)

    out = _core(x2, wconv, bconv, w1, b1, w2, b2, wfc_p, bfc, b)
    return out.reshape(B_pad, NCP)[:B, :NC]


def kernel(conv_w, conv_b, w_ih1, w_hh1, b_ih1, b_hh1,
           w_ih2, w_hh2, b_ih2, b_hh2, fc_w, fc_b, x):
    args = (conv_w, conv_b, w_ih1, w_hh1, b_ih1, b_hh1,
            w_ih2, w_hh2, b_ih2, b_hh2, fc_w, fc_b)
    x_bf = x.astype(jnp.bfloat16)
    devs = jax.devices()
    B = x.shape[0]
    if len(devs) < 2 or B % 2:
        return _prep_and_run(*args, x_bf)
    mesh = Mesh(np.array(devs[:2]), ("d",))
    return shard_map(_prep_and_run, mesh=mesh,
                     in_specs=tuple([P()] * 12 + [P("d")]),
                     out_specs=P("d"), check_rep=False)(*args, x_bf)
